# baseline (speedup 1.0000x reference)
import jax
import jax.numpy as jnp
from jax.experimental import pallas as pl

N = 16384
K = 512
BN = 1024


def _score_block(t_ref, o_ref):
    d = t_ref[0] + t_ref[1] - t_ref[2]
    o_ref[...] = -jnp.sqrt(jnp.sum(d * d, axis=1))


def kernel(triples):
    grid = (N // BN,)
    return pl.pallas_call(
        _score_block,
        grid=grid,
        in_specs=[pl.BlockSpec((3, BN, K), lambda i: (0, i, 0))],
        out_specs=pl.BlockSpec((BN,), lambda i: (i,)),
        out_shape=jax.ShapeDtypeStruct((N,), jnp.float32),
    )(triples)


# speedup vs baseline: 1.0095x; 1.0095x over previous
import jax
import jax.numpy as jnp
from jax.experimental import pallas as pl

N = 16384
K = 512
BN = 2048


def _score_block(t_ref, o_ref):
    d = t_ref[0] + t_ref[1] - t_ref[2]
    o_ref[...] = -jnp.sqrt(jnp.sum(d * d, axis=1))


def kernel(triples):
    grid = (N // BN,)
    return pl.pallas_call(
        _score_block,
        grid=grid,
        in_specs=[pl.BlockSpec((3, BN, K), lambda i: (0, i, 0))],
        out_specs=pl.BlockSpec((BN,), lambda i: (i,)),
        out_shape=jax.ShapeDtypeStruct((N,), jnp.float32),
    )(triples)
